# baseline (speedup 1.0000x reference)
import jax
import jax.numpy as jnp
from jax import lax
from jax.experimental import pallas as pl
from jax.experimental.pallas import tpu as pltpu
from jax.experimental.pallas import tpu_sc as plsc

N_ROWS = 32768
N_EXP = 64
KK = 8
NUM_CORES = 2
NUM_SUBCORES = 16
NW = NUM_CORES * NUM_SUBCORES

SC_ROWS = 16384
TC_ROWS = N_ROWS - SC_ROWS
ROWS_PER_W = SC_ROWS // NW
CHUNK = 128
NCHUNK = ROWS_PER_W // CHUNK

BLK = 2048
NEG = -3.0e38


def _sort16(x):
    return lax.sort(x, dimension=0, is_stable=False)


def _top16(a, b):
    return _sort16(jnp.maximum(a, lax.rev(b, (0,))))


def _sc_body(x_hbm, o_hbm, xb0, xb1, ob0, ob1, si0, si1, so0, so1):
    wid = lax.axis_index("s") * NUM_CORES + lax.axis_index("c")
    base = wid * ROWS_PER_W
    lane = lax.iota(jnp.int32, 16)
    xbufs, obufs, sins, souts = (xb0, xb1), (ob0, ob1), (si0, si1), (so0, so1)

    def start_in(c, b):
        return pltpu.async_copy(
            x_hbm.at[pl.ds(base + c * CHUNK, CHUNK)], xbufs[b], sins[b]
        )

    def compute_chunk(xbuf, obuf):
        @plsc.parallel_loop(0, CHUNK, step=1, unroll=4)
        def row_body(r):
            v0 = xbuf[r, pl.ds(0, 16)]
            v1 = xbuf[r, pl.ds(16, 16)]
            v2 = xbuf[r, pl.ds(32, 16)]
            v3 = xbuf[r, pl.ds(48, 16)]
            t01 = _top16(_sort16(v0), _sort16(v1))
            t23 = _top16(_sort16(v2), _sort16(v3))
            t = _top16(t01, t23)
            m = jnp.max(t)
            thr = jnp.sum(jnp.where(lane == KK, t, 0.0))
            e = jnp.exp(t - m)
            denom = jnp.sum(jnp.where(lane >= KK, e, 0.0))
            ones = jnp.full((16,), 1.0, jnp.float32)
            recipv = ones / (ones * denom)
            for j, v in enumerate((v0, v1, v2, v3)):
                w = jnp.where(v >= thr, jnp.exp(v - m) * recipv, 0.0)
                obuf[r, pl.ds(j * 16, 16)] = w

    pending_in = [None] * NCHUNK
    pending_out = [None] * NCHUNK
    pending_in[0] = start_in(0, 0)
    for c in range(NCHUNK):
        b = c & 1
        if c + 1 < NCHUNK:
            pending_in[c + 1] = start_in(c + 1, 1 - b)
        pending_in[c].wait()
        if c >= 2:
            pending_out[c - 2].wait()
        compute_chunk(xbufs[b], obufs[b])
        pending_out[c] = pltpu.async_copy(
            obufs[b], o_hbm.at[pl.ds(base + c * CHUNK, CHUNK)], souts[b]
        )
    pending_out[NCHUNK - 2].wait()
    pending_out[NCHUNK - 1].wait()


def _tc_body(x_ref, o_ref):
    x = x_ref[...]
    work = x
    mk = jnp.max(work, axis=1, keepdims=True)
    m = mk
    d = jnp.ones_like(mk)
    for k in range(7):
        work = jnp.where(work == mk, NEG, work)
        mk = jnp.max(work, axis=1, keepdims=True)
        d = d + jnp.exp(mk - m)
    thr = mk
    o_ref[...] = jnp.where(x >= thr, jnp.exp(x - m) / d, 0.0)


@jax.jit
def kernel(logits):
    mesh = plsc.VectorSubcoreMesh(core_axis_name="c", subcore_axis_name="s")
    sc_out = pl.kernel(
        _sc_body,
        out_type=jax.ShapeDtypeStruct((SC_ROWS, N_EXP), jnp.float32),
        mesh=mesh,
        scratch_types=[pltpu.VMEM((CHUNK, N_EXP), jnp.float32)] * 4
        + [pltpu.SemaphoreType.DMA] * 4,
        compiler_params=pltpu.CompilerParams(needs_layout_passes=False),
    )(logits[:SC_ROWS])
    tc_out = pl.pallas_call(
        _tc_body,
        grid=(TC_ROWS // BLK,),
        in_specs=[pl.BlockSpec((BLK, N_EXP), lambda i: (i, 0))],
        out_specs=pl.BlockSpec((BLK, N_EXP), lambda i: (i, 0)),
        out_shape=jax.ShapeDtypeStruct((TC_ROWS, N_EXP), jnp.float32),
    )(logits[SC_ROWS:])
    return jnp.concatenate([sc_out, tc_out], axis=0)
